# Initial kernel scaffold; baseline (speedup 1.0000x reference)
import jax
import jax.numpy as jnp
from jax.experimental import pallas as pl


def kernel(x, edge_index, graph_ids, self_feat, W1, b1, W2, b2, Wbil, fc1_W, fc1_b, fc2_W, fc2_b, fc3_W, fc3_b, bn1_g, bn1_b, bn2_g, bn2_b):
    raise NotImplementedError("write your pallas kernel here")


# speedup vs baseline: 15.2679x; 15.2679x over previous
import functools

import jax
import jax.numpy as jnp
from jax import lax
from jax.experimental import pallas as pl
from jax.experimental.pallas import tpu as pltpu
from jax.experimental.pallas import tpu_sc as plsc

_EB = 80
_ROW_BLK = 1000


@functools.cache
def _sc_aggregate(n, e, d, dt=jnp.float32):
    info = plsc.get_sparse_core_info()
    nc, ns = info.num_cores, info.num_subcores
    nw = nc * ns
    steps = e // nw // _EB
    chunk = 25
    nchunks = steps // chunk
    assert steps % chunk == 0 and chunk % 2 == 1
    rpt = (n // ns) // 8 * 8
    tail = n - rpt * ns
    zrows = 48
    lanes = 32 if dt == jnp.bfloat16 else 16
    mesh = plsc.VectorSubcoreMesh(core_axis_name="c", subcore_axis_name="s")

    @functools.partial(
        pl.kernel,
        out_type=(jax.ShapeDtypeStruct((n, d), dt),
                  jax.ShapeDtypeStruct((n, d), dt)),
        mesh=mesh,
        scratch_types=[
            pltpu.VMEM((chunk, _EB), jnp.int32),
            pltpu.VMEM((chunk, _EB), jnp.int32),
            pltpu.VMEM((_EB, d), dt),
            pltpu.VMEM((_EB, d), dt),
            pltpu.VMEM((zrows, d), dt),
            pltpu.VMEM_SHARED((n, d), dt),
            pltpu.SemaphoreType.DMA,
            pltpu.SemaphoreType.DMA,
        ],
        compiler_params=pltpu.CompilerParams(use_tc_tiling_on_sc=False),
    )
    def agg(y_hbm, srcg_hbm, dstg_hbm, out0, out1,
            src_v, dst_v, rows_a, rows_b, zbuf, acc_sh, sem_a, sem_b):
        c = lax.axis_index("c")
        s = lax.axis_index("s")
        wid = c * ns + s

        def _zrow(r, carry):
            for k in range(d // lanes):
                zbuf[r, pl.ds(k * lanes, lanes)] = jnp.zeros((lanes,), dt)
            return carry
        lax.fori_loop(0, zrows, _zrow, 0)

        def _zcp(k, carry):
            pltpu.sync_copy(zbuf, acc_sh.at[pl.ds(s * rpt + k * zrows, zrows)])
            return carry
        lax.fori_loop(0, rpt // zrows, _zcp, 0)

        @pl.when(s == ns - 1)
        def _():
            pltpu.sync_copy(zbuf.at[pl.ds(0, tail)],
                            acc_sh.at[pl.ds(ns * rpt, tail)])
        plsc.subcore_barrier()

        def _chunk(ci, carry):
            pltpu.sync_copy(srcg_hbm.at[wid, pl.ds(ci * chunk, chunk)], src_v)
            pltpu.sync_copy(dstg_hbm.at[wid, pl.ds(ci * chunk, chunk)], dst_v)
            pltpu.make_async_copy(y_hbm.at[src_v.at[0]], rows_a, sem_a).start()

            def _step(j, carry2):
                jj = 2 * j
                pltpu.make_async_copy(
                    y_hbm.at[src_v.at[jj + 1]], rows_b, sem_b).start()
                pltpu.make_async_copy(
                    y_hbm.at[src_v.at[jj]], rows_a, sem_a).wait()
                pltpu.sync_copy(rows_a, acc_sh.at[dst_v.at[jj]], add=True)
                pltpu.make_async_copy(
                    y_hbm.at[src_v.at[jj + 2]], rows_a, sem_a).start()
                pltpu.make_async_copy(
                    y_hbm.at[src_v.at[jj + 1]], rows_b, sem_b).wait()
                pltpu.sync_copy(rows_b, acc_sh.at[dst_v.at[jj + 1]], add=True)
                return carry2
            lax.fori_loop(0, (chunk - 1) // 2, _step, 0)

            pltpu.make_async_copy(
                y_hbm.at[src_v.at[chunk - 1]], rows_a, sem_a).wait()
            pltpu.sync_copy(rows_a, acc_sh.at[dst_v.at[chunk - 1]], add=True)
            return carry
        lax.fori_loop(0, nchunks, _chunk, 0)

        plsc.subcore_barrier()
        sl = pl.ds(s * rpt, rpt)
        sl_t = pl.ds(ns * rpt, tail)

        @pl.when(c == 0)
        def _():
            pltpu.sync_copy(acc_sh.at[sl], out0.at[sl])

            @pl.when(s == ns - 1)
            def _():
                pltpu.sync_copy(acc_sh.at[sl_t], out0.at[sl_t])

        @pl.when(c == 1)
        def _():
            pltpu.sync_copy(acc_sh.at[sl], out1.at[sl])

            @pl.when(s == ns - 1)
            def _():
                pltpu.sync_copy(acc_sh.at[sl_t], out1.at[sl_t])

    return agg


def _dot(a, b):
    return jnp.dot(a, b, preferred_element_type=jnp.float32,
                   precision=lax.Precision.DEFAULT)


def _dot_exact(a, b):
    return jnp.dot(a, b, preferred_element_type=jnp.float32,
                   precision=lax.Precision.HIGHEST)


def _mm_body(x_ref, w_ref, b_ref, o_ref):
    o_ref[...] = (_dot(x_ref[...], w_ref[...]) + b_ref[...]).astype(o_ref.dtype)


def _combine_mm_body(dcol, p0_ref, p1_ref, y_ref, w_ref, b_ref, o_ref):
    acc = p0_ref[...].astype(jnp.float32) + p1_ref[...].astype(jnp.float32)
    deg = acc[:, dcol:dcol + 1]
    mean = acc / jnp.maximum(deg, 1.0)
    h = jnp.maximum(
        jnp.where(deg > 0, mean, y_ref[...].astype(jnp.float32)), 0.0)
    o_ref[...] = (_dot(h, w_ref[...]) + b_ref[...]).astype(o_ref.dtype)


def _readout_head_body(nsteps, dcol, p0_ref, p1_ref, y_ref, gid_ref,
                       self_ref, wbil_ref, f1w_ref, f1b_ref, f2w_ref,
                       f2b_ref, f3w_ref, f3b_ref, g1_ref, bb1_ref,
                       g2_ref, bb2_ref, o_ref, m_acc):
    i = pl.program_id(0)

    @pl.when(i == 0)
    def _():
        m_acc[...] = jnp.zeros_like(m_acc)

    acc = p0_ref[...].astype(jnp.float32) + p1_ref[...].astype(jnp.float32)
    deg = acc[:, dcol:dcol + 1]
    mean = acc / jnp.maximum(deg, 1.0)
    h2 = jnp.maximum(
        jnp.where(deg > 0, mean, y_ref[...].astype(jnp.float32)), 0.0)

    gids = gid_ref[0, 0, :]
    seg = lax.broadcasted_iota(jnp.int32, (16, h2.shape[0]), 0)
    oht = (gids[None, :] == seg).astype(jnp.float32)
    m_acc[...] += _dot_exact(oht, h2)

    @pl.when(i == nsteps - 1)
    def _():
        m = m_acc[...]
        cnt = m[:, dcol:dcol + 1]
        hg = m[:, :dcol] / jnp.maximum(cnt, 1.0)
        a = _dot(hg, wbil_ref[...]) * self_ref[...]
        z = _dot(a, f1w_ref[...]) + f1b_ref[...]
        mu = jnp.mean(z, axis=0, keepdims=True)
        var = jnp.mean((z - mu) ** 2, axis=0, keepdims=True)
        z = g1_ref[...] * (z - mu) / jnp.sqrt(var + 1e-5) + bb1_ref[...]
        z = jnp.maximum(z, 0.0)
        z = _dot(z, f2w_ref[...]) + f2b_ref[...]
        mu = jnp.mean(z, axis=0, keepdims=True)
        var = jnp.mean((z - mu) ** 2, axis=0, keepdims=True)
        z = g2_ref[...] * (z - mu) / jnp.sqrt(var + 1e-5) + bb2_ref[...]
        z = jnp.maximum(z, 0.0)
        o_ref[...] = _dot(z, f3w_ref[...]) + f3b_ref[...]


def _full(shape):
    return pl.BlockSpec(shape, lambda i: tuple(0 for _ in shape))


def kernel(x, edge_index, graph_ids, self_feat, W1, b1, W2, b2, Wbil,
           fc1_W, fc1_b, fc2_W, fc2_b, fc3_W, fc3_b,
           bn1_g, bn1_b, bn2_g, bn2_b):
    n, din = x.shape
    e = edge_index.shape[1]
    d1 = W1.shape[1]
    d1p = ((d1 + 1 + 31) // 32) * 32
    d2 = W2.shape[1]
    d2p = ((d2 + 1 + 31) // 32) * 32
    agg_dt = jnp.bfloat16
    nblk = n // _ROW_BLK

    src2 = edge_index[0].astype(jnp.int32).reshape(32, e // 32 // _EB, _EB)
    dst2 = edge_index[1].astype(jnp.int32).reshape(32, e // 32 // _EB, _EB)
    gid3 = graph_ids.astype(jnp.int32).reshape(nblk, 1, _ROW_BLK)

    w1p = jnp.zeros((din, d1p), jnp.float32).at[:, :d1].set(W1)
    b1p = jnp.zeros((1, d1p), jnp.float32).at[0, :d1].set(b1).at[0, d1].set(1.0)
    w2p = jnp.zeros((d1p, d2p), jnp.float32).at[:d1, :d2].set(W2)
    b2p = jnp.zeros((1, d2p), jnp.float32).at[0, :d2].set(b2).at[0, d2].set(1.0)

    row = lambda shape: pl.BlockSpec(shape, lambda i: (i, 0))

    y1 = pl.pallas_call(
        _mm_body,
        grid=(nblk,),
        in_specs=[row((_ROW_BLK, din)), _full((din, d1p)), _full((1, d1p))],
        out_specs=row((_ROW_BLK, d1p)),
        out_shape=jax.ShapeDtypeStruct((n, d1p), agg_dt),
    )(x, w1p, b1p)

    p0, p1 = _sc_aggregate(n, e, d1p, agg_dt)(y1, src2, dst2)

    y2 = pl.pallas_call(
        functools.partial(_combine_mm_body, d1),
        grid=(nblk,),
        in_specs=[row((_ROW_BLK, d1p)), row((_ROW_BLK, d1p)),
                  row((_ROW_BLK, d1p)), _full((d1p, d2p)), _full((1, d2p))],
        out_specs=row((_ROW_BLK, d2p)),
        out_shape=jax.ShapeDtypeStruct((n, d2p), agg_dt),
    )(p0, p1, y1, w2p, b2p)

    q0, q1 = _sc_aggregate(n, e, d2p, agg_dt)(y2, src2, dst2)

    out = pl.pallas_call(
        functools.partial(_readout_head_body, nblk, d2),
        grid=(nblk,),
        in_specs=[row((_ROW_BLK, d2p)), row((_ROW_BLK, d2p)),
                  row((_ROW_BLK, d2p)),
                  pl.BlockSpec((1, 1, _ROW_BLK), lambda i: (i, 0, 0)),
                  _full(self_feat.shape), _full(Wbil.shape),
                  _full(fc1_W.shape), _full((1, fc1_b.shape[0])),
                  _full(fc2_W.shape), _full((1, fc2_b.shape[0])),
                  _full(fc3_W.shape), _full((1, fc3_b.shape[0])),
                  _full((1, bn1_g.shape[0])), _full((1, bn1_b.shape[0])),
                  _full((1, bn2_g.shape[0])), _full((1, bn2_b.shape[0]))],
        out_specs=_full((16, fc3_W.shape[1])),
        out_shape=jax.ShapeDtypeStruct((16, fc3_W.shape[1]), jnp.float32),
        scratch_shapes=[pltpu.VMEM((16, d2p), jnp.float32)],
    )(q0, q1, y2, gid3, self_feat, Wbil,
      fc1_W, fc1_b.reshape(1, -1), fc2_W, fc2_b.reshape(1, -1),
      fc3_W, fc3_b.reshape(1, -1),
      bn1_g.reshape(1, -1), bn1_b.reshape(1, -1),
      bn2_g.reshape(1, -1), bn2_b.reshape(1, -1))
    return out


# speedup vs baseline: 19.4110x; 1.2714x over previous
import functools

import jax
import jax.numpy as jnp
from jax import lax
from jax.experimental import pallas as pl
from jax.experimental.pallas import tpu as pltpu
from jax.experimental.pallas import tpu_sc as plsc

_EB = 80
_ROW_BLK = 1000


@functools.cache
def _sc_aggregate(n, e, d, dt=jnp.float32):
    info = plsc.get_sparse_core_info()
    nc, ns = info.num_cores, info.num_subcores
    nw = nc * ns
    steps = e // nw // _EB
    assert steps % 4 == 1
    rpt = (n // ns) // 8 * 8
    tail = n - rpt * ns
    zrows = 48
    lanes = 32 if dt == jnp.bfloat16 else 16
    mesh = plsc.VectorSubcoreMesh(core_axis_name="c", subcore_axis_name="s")

    @functools.partial(
        pl.kernel,
        out_type=(jax.ShapeDtypeStruct((n, d), dt),
                  jax.ShapeDtypeStruct((n, d), dt)),
        mesh=mesh,
        scratch_types=[
            pltpu.VMEM((steps, _EB), jnp.int32),
            pltpu.VMEM((steps, _EB), jnp.int32),
            pltpu.VMEM((_EB, d), dt),
            pltpu.VMEM((_EB, d), dt),
            pltpu.VMEM((_EB, d), dt),
            pltpu.VMEM((_EB, d), dt),
            pltpu.VMEM((zrows, d), dt),
            pltpu.VMEM_SHARED((n, d), dt),
            pltpu.SemaphoreType.DMA,
            pltpu.SemaphoreType.DMA,
            pltpu.SemaphoreType.DMA,
            pltpu.SemaphoreType.DMA,
            pltpu.SemaphoreType.DMA,
            pltpu.SemaphoreType.DMA,
            pltpu.SemaphoreType.DMA,
            pltpu.SemaphoreType.DMA,
        ],
        compiler_params=pltpu.CompilerParams(use_tc_tiling_on_sc=False),
    )
    def agg(y_hbm, srcg_hbm, dstg_hbm, out0, out1,
            src_v, dst_v, r0, r1, r2, r3, zbuf, acc_sh,
            g0, g1, g2, g3, s0, s1, s2, s3):
        rows = (r0, r1, r2, r3)
        gsem = (g0, g1, g2, g3)
        ssem = (s0, s1, s2, s3)
        c = lax.axis_index("c")
        s = lax.axis_index("s")
        wid = c * ns + s

        def _zrow(r, carry):
            for k in range(d // lanes):
                zbuf[r, pl.ds(k * lanes, lanes)] = jnp.zeros((lanes,), dt)
            return carry
        lax.fori_loop(0, zrows, _zrow, 0)

        def _zcp(k, carry):
            pltpu.sync_copy(zbuf, acc_sh.at[pl.ds(s * rpt + k * zrows, zrows)])
            return carry
        lax.fori_loop(0, rpt // zrows, _zcp, 0)

        @pl.when(s == ns - 1)
        def _():
            pltpu.sync_copy(zbuf.at[pl.ds(0, tail)],
                            acc_sh.at[pl.ds(ns * rpt, tail)])
        plsc.subcore_barrier()

        pltpu.sync_copy(srcg_hbm.at[wid], src_v)
        pltpu.sync_copy(dstg_hbm.at[wid], dst_v)

        def g_start(step, b):
            pltpu.async_copy(y_hbm.at[src_v.at[step]], rows[b], gsem[b])

        def g_wait(step, b):
            pltpu.make_async_copy(
                y_hbm.at[src_v.at[step]], rows[b], gsem[b]).wait()

        def s_start(step, b):
            pltpu.async_copy(
                rows[b], acc_sh.at[dst_v.at[step]], ssem[b], add=True)

        def s_wait(step, b):
            pltpu.make_async_copy(
                rows[b], acc_sh.at[dst_v.at[step]], ssem[b]).wait()

        for k in range(3):
            g_start(k, k)
        g_wait(0, 0)
        s_start(0, 0)
        g_start(3, 3)

        def _grp(g, carry):
            base = 1 + 4 * g
            for k in range(4):
                st = base + k
                b = (1 + k) % 4
                bn = k % 4
                g_wait(st, b)
                s_start(st, b)
                s_wait(st - 1, bn)
                g_start(st + 3, bn)
            return carry
        lax.fori_loop(0, (steps - 5) // 4, _grp, 0)

        t = steps - 4
        g_wait(t, (t + 0) % 4); s_start(t, t % 4)
        s_wait(t - 1, (t + 3) % 4); g_start(t + 3, (t + 3) % 4)
        for k in range(1, 4):
            g_wait(t + k, (t + k) % 4)
            s_start(t + k, (t + k) % 4)
        for k in range(0, 4):
            s_wait(t + k, (t + k) % 4)

        plsc.subcore_barrier()
        sl = pl.ds(s * rpt, rpt)
        sl_t = pl.ds(ns * rpt, tail)

        @pl.when(c == 0)
        def _():
            pltpu.sync_copy(acc_sh.at[sl], out0.at[sl])

            @pl.when(s == ns - 1)
            def _():
                pltpu.sync_copy(acc_sh.at[sl_t], out0.at[sl_t])

        @pl.when(c == 1)
        def _():
            pltpu.sync_copy(acc_sh.at[sl], out1.at[sl])

            @pl.when(s == ns - 1)
            def _():
                pltpu.sync_copy(acc_sh.at[sl_t], out1.at[sl_t])

    return agg


def _dot(a, b):
    return jnp.dot(a, b, preferred_element_type=jnp.float32,
                   precision=lax.Precision.DEFAULT)


def _dot_exact(a, b):
    return jnp.dot(a, b, preferred_element_type=jnp.float32,
                   precision=lax.Precision.HIGHEST)


def _mm_body(x_ref, w_ref, b_ref, o_ref):
    o_ref[...] = (_dot(x_ref[...], w_ref[...]) + b_ref[...]).astype(o_ref.dtype)


def _combine_mm_body(dcol, p0_ref, p1_ref, y_ref, w_ref, b_ref, o_ref):
    acc = p0_ref[...].astype(jnp.float32) + p1_ref[...].astype(jnp.float32)
    deg = acc[:, dcol:dcol + 1]
    mean = acc / jnp.maximum(deg, 1.0)
    h = jnp.maximum(
        jnp.where(deg > 0, mean, y_ref[...].astype(jnp.float32)), 0.0)
    o_ref[...] = (_dot(h, w_ref[...]) + b_ref[...]).astype(o_ref.dtype)


def _readout_head_body(nsteps, dcol, p0_ref, p1_ref, y_ref, gid_ref,
                       self_ref, wbil_ref, f1w_ref, f1b_ref, f2w_ref,
                       f2b_ref, f3w_ref, f3b_ref, g1_ref, bb1_ref,
                       g2_ref, bb2_ref, o_ref, m_acc):
    i = pl.program_id(0)

    @pl.when(i == 0)
    def _():
        m_acc[...] = jnp.zeros_like(m_acc)

    acc = p0_ref[...].astype(jnp.float32) + p1_ref[...].astype(jnp.float32)
    deg = acc[:, dcol:dcol + 1]
    mean = acc / jnp.maximum(deg, 1.0)
    h2 = jnp.maximum(
        jnp.where(deg > 0, mean, y_ref[...].astype(jnp.float32)), 0.0)

    gids = gid_ref[0, 0, :]
    seg = lax.broadcasted_iota(jnp.int32, (16, h2.shape[0]), 0)
    oht = (gids[None, :] == seg).astype(jnp.float32)
    m_acc[...] += _dot_exact(oht, h2)

    @pl.when(i == nsteps - 1)
    def _():
        m = m_acc[...]
        cnt = m[:, dcol:dcol + 1]
        hg = m[:, :dcol] / jnp.maximum(cnt, 1.0)
        a = _dot(hg, wbil_ref[...]) * self_ref[...]
        z = _dot(a, f1w_ref[...]) + f1b_ref[...]
        mu = jnp.mean(z, axis=0, keepdims=True)
        var = jnp.mean((z - mu) ** 2, axis=0, keepdims=True)
        z = g1_ref[...] * (z - mu) / jnp.sqrt(var + 1e-5) + bb1_ref[...]
        z = jnp.maximum(z, 0.0)
        z = _dot(z, f2w_ref[...]) + f2b_ref[...]
        mu = jnp.mean(z, axis=0, keepdims=True)
        var = jnp.mean((z - mu) ** 2, axis=0, keepdims=True)
        z = g2_ref[...] * (z - mu) / jnp.sqrt(var + 1e-5) + bb2_ref[...]
        z = jnp.maximum(z, 0.0)
        o_ref[...] = _dot(z, f3w_ref[...]) + f3b_ref[...]


def _full(shape):
    return pl.BlockSpec(shape, lambda i: tuple(0 for _ in shape))


def kernel(x, edge_index, graph_ids, self_feat, W1, b1, W2, b2, Wbil,
           fc1_W, fc1_b, fc2_W, fc2_b, fc3_W, fc3_b,
           bn1_g, bn1_b, bn2_g, bn2_b):
    n, din = x.shape
    e = edge_index.shape[1]
    d1 = W1.shape[1]
    d1p = ((d1 + 1 + 31) // 32) * 32
    d2 = W2.shape[1]
    d2p = ((d2 + 1 + 31) // 32) * 32
    agg_dt = jnp.bfloat16
    nblk = n // _ROW_BLK

    src2 = edge_index[0].astype(jnp.int32).reshape(32, e // 32 // _EB, _EB)
    dst2 = edge_index[1].astype(jnp.int32).reshape(32, e // 32 // _EB, _EB)
    gid3 = graph_ids.astype(jnp.int32).reshape(nblk, 1, _ROW_BLK)

    w1p = jnp.zeros((din, d1p), jnp.float32).at[:, :d1].set(W1)
    b1p = jnp.zeros((1, d1p), jnp.float32).at[0, :d1].set(b1).at[0, d1].set(1.0)
    w2p = jnp.zeros((d1p, d2p), jnp.float32).at[:d1, :d2].set(W2)
    b2p = jnp.zeros((1, d2p), jnp.float32).at[0, :d2].set(b2).at[0, d2].set(1.0)

    row = lambda shape: pl.BlockSpec(shape, lambda i: (i, 0))

    y1 = pl.pallas_call(
        _mm_body,
        grid=(nblk,),
        in_specs=[row((_ROW_BLK, din)), _full((din, d1p)), _full((1, d1p))],
        out_specs=row((_ROW_BLK, d1p)),
        out_shape=jax.ShapeDtypeStruct((n, d1p), agg_dt),
    )(x, w1p, b1p)

    p0, p1 = _sc_aggregate(n, e, d1p, agg_dt)(y1, src2, dst2)

    y2 = pl.pallas_call(
        functools.partial(_combine_mm_body, d1),
        grid=(nblk,),
        in_specs=[row((_ROW_BLK, d1p)), row((_ROW_BLK, d1p)),
                  row((_ROW_BLK, d1p)), _full((d1p, d2p)), _full((1, d2p))],
        out_specs=row((_ROW_BLK, d2p)),
        out_shape=jax.ShapeDtypeStruct((n, d2p), agg_dt),
    )(p0, p1, y1, w2p, b2p)

    q0, q1 = _sc_aggregate(n, e, d2p, agg_dt)(y2, src2, dst2)

    out = pl.pallas_call(
        functools.partial(_readout_head_body, nblk, d2),
        grid=(nblk,),
        in_specs=[row((_ROW_BLK, d2p)), row((_ROW_BLK, d2p)),
                  row((_ROW_BLK, d2p)),
                  pl.BlockSpec((1, 1, _ROW_BLK), lambda i: (i, 0, 0)),
                  _full(self_feat.shape), _full(Wbil.shape),
                  _full(fc1_W.shape), _full((1, fc1_b.shape[0])),
                  _full(fc2_W.shape), _full((1, fc2_b.shape[0])),
                  _full(fc3_W.shape), _full((1, fc3_b.shape[0])),
                  _full((1, bn1_g.shape[0])), _full((1, bn1_b.shape[0])),
                  _full((1, bn2_g.shape[0])), _full((1, bn2_b.shape[0]))],
        out_specs=_full((16, fc3_W.shape[1])),
        out_shape=jax.ShapeDtypeStruct((16, fc3_W.shape[1]), jnp.float32),
        scratch_shapes=[pltpu.VMEM((16, d2p), jnp.float32)],
    )(q0, q1, y2, gid3, self_feat, Wbil,
      fc1_W, fc1_b.reshape(1, -1), fc2_W, fc2_b.reshape(1, -1),
      fc3_W, fc3_b.reshape(1, -1),
      bn1_g.reshape(1, -1), bn1_b.reshape(1, -1),
      bn2_g.reshape(1, -1), bn2_b.reshape(1, -1))
    return out


# speedup vs baseline: 21.0617x; 1.0850x over previous
import functools

import jax
import jax.numpy as jnp
from jax import lax
from jax.experimental import pallas as pl
from jax.experimental.pallas import tpu as pltpu
from jax.experimental.pallas import tpu_sc as plsc

_EB = 80
_ROW_BLK = 1000


@functools.cache
def _sc_aggregate(n, e, d, dt=jnp.float32):
    info = plsc.get_sparse_core_info()
    nc, ns = info.num_cores, info.num_subcores
    nw = nc * ns
    steps = e // nw // _EB
    assert steps % 4 == 1
    rpt = (n // ns) // 8 * 8
    tail = n - rpt * ns
    zrows = 48
    lanes = 32 if dt == jnp.bfloat16 else 16
    mesh = plsc.VectorSubcoreMesh(core_axis_name="c", subcore_axis_name="s")

    @functools.partial(
        pl.kernel,
        out_type=(jax.ShapeDtypeStruct((n, d), dt),
                  jax.ShapeDtypeStruct((n, d), dt)),
        mesh=mesh,
        scratch_types=[
            pltpu.VMEM((steps, _EB), jnp.int32),
            pltpu.VMEM((steps, _EB), jnp.int32),
            pltpu.VMEM((_EB, d), dt),
            pltpu.VMEM((_EB, d), dt),
            pltpu.VMEM((_EB, d), dt),
            pltpu.VMEM((_EB, d), dt),
            pltpu.VMEM((zrows, d), dt),
            pltpu.VMEM_SHARED((n, d), dt),
            pltpu.SemaphoreType.DMA,
            pltpu.SemaphoreType.DMA,
            pltpu.SemaphoreType.DMA,
            pltpu.SemaphoreType.DMA,
            pltpu.SemaphoreType.DMA,
            pltpu.SemaphoreType.DMA,
            pltpu.SemaphoreType.DMA,
            pltpu.SemaphoreType.DMA,
        ],
        compiler_params=pltpu.CompilerParams(use_tc_tiling_on_sc=False),
    )
    def agg(y_hbm, e4_hbm, out0, out1,
            src_v, dst_v, r0, r1, r2, r3, zbuf, acc_sh,
            g0, g1, g2, g3, s0, s1, s2, s3):
        rows = (r0, r1, r2, r3)
        gsem = (g0, g1, g2, g3)
        ssem = (s0, s1, s2, s3)
        c = lax.axis_index("c")
        s = lax.axis_index("s")
        wid = c * ns + s

        def _zrow(r, carry):
            for k in range(d // lanes):
                zbuf[r, pl.ds(k * lanes, lanes)] = jnp.zeros((lanes,), dt)
            return carry
        lax.fori_loop(0, zrows, _zrow, 0)

        def _zcp(k, carry):
            pltpu.sync_copy(zbuf, acc_sh.at[pl.ds(s * rpt + k * zrows, zrows)])
            return carry
        lax.fori_loop(0, rpt // zrows, _zcp, 0)

        @pl.when(s == ns - 1)
        def _():
            pltpu.sync_copy(zbuf.at[pl.ds(0, tail)],
                            acc_sh.at[pl.ds(ns * rpt, tail)])
        plsc.subcore_barrier()

        pltpu.sync_copy(e4_hbm.at[0, wid], src_v)
        pltpu.sync_copy(e4_hbm.at[1, wid], dst_v)

        def g_start(step, b):
            pltpu.async_copy(y_hbm.at[src_v.at[step]], rows[b], gsem[b])

        def g_wait(step, b):
            pltpu.make_async_copy(
                y_hbm.at[src_v.at[step]], rows[b], gsem[b]).wait()

        def s_start(step, b):
            pltpu.async_copy(
                rows[b], acc_sh.at[dst_v.at[step]], ssem[b], add=True)

        def s_wait(step, b):
            pltpu.make_async_copy(
                rows[b], acc_sh.at[dst_v.at[step]], ssem[b]).wait()

        for k in range(3):
            g_start(k, k)
        g_wait(0, 0)
        s_start(0, 0)
        g_start(3, 3)

        def _grp(g, carry):
            base = 1 + 4 * g
            for k in range(4):
                st = base + k
                b = (1 + k) % 4
                bn = k % 4
                g_wait(st, b)
                s_start(st, b)
                s_wait(st - 1, bn)
                g_start(st + 3, bn)
            return carry
        lax.fori_loop(0, (steps - 5) // 4, _grp, 0)

        t = steps - 4
        g_wait(t, (t + 0) % 4); s_start(t, t % 4)
        s_wait(t - 1, (t + 3) % 4); g_start(t + 3, (t + 3) % 4)
        for k in range(1, 4):
            g_wait(t + k, (t + k) % 4)
            s_start(t + k, (t + k) % 4)
        for k in range(0, 4):
            s_wait(t + k, (t + k) % 4)

        plsc.subcore_barrier()
        sl = pl.ds(s * rpt, rpt)
        sl_t = pl.ds(ns * rpt, tail)

        @pl.when(c == 0)
        def _():
            pltpu.sync_copy(acc_sh.at[sl], out0.at[sl])

            @pl.when(s == ns - 1)
            def _():
                pltpu.sync_copy(acc_sh.at[sl_t], out0.at[sl_t])

        @pl.when(c == 1)
        def _():
            pltpu.sync_copy(acc_sh.at[sl], out1.at[sl])

            @pl.when(s == ns - 1)
            def _():
                pltpu.sync_copy(acc_sh.at[sl_t], out1.at[sl_t])

    return agg


def _dot(a, b):
    return jnp.dot(a, b, preferred_element_type=jnp.float32,
                   precision=lax.Precision.DEFAULT)


def _dot_exact(a, b):
    return jnp.dot(a, b, preferred_element_type=jnp.float32,
                   precision=lax.Precision.HIGHEST)


def _pad_ones(z, width):
    rows = z.shape[0]
    ones = jnp.ones((rows, 1), jnp.float32)
    zeros = jnp.zeros((rows, width - z.shape[1] - 1), jnp.float32)
    return jnp.concatenate([z, ones, zeros], axis=1)


def _mm_body(x_ref, w_ref, b_ref, o_ref):
    y = _dot(x_ref[...], w_ref[...]) + b_ref[...]
    o_ref[...] = _pad_ones(y, o_ref.shape[1]).astype(o_ref.dtype)


def _combine_mm_body(dcol, p0_ref, p1_ref, y_ref, w_ref, b_ref, o_ref):
    acc = p0_ref[...].astype(jnp.float32) + p1_ref[...].astype(jnp.float32)
    deg = acc[:, dcol:dcol + 1]
    mean = acc / jnp.maximum(deg, 1.0)
    h = jnp.maximum(
        jnp.where(deg > 0, mean, y_ref[...].astype(jnp.float32)), 0.0)
    z = _dot(h[:, :dcol], w_ref[...]) + b_ref[...]
    o_ref[...] = _pad_ones(z, o_ref.shape[1]).astype(o_ref.dtype)


def _readout_head_body(nsteps, dcol, p0_ref, p1_ref, y_ref, gid_ref,
                       self_ref, wbil_ref, f1w_ref, f1b_ref, f2w_ref,
                       f2b_ref, f3w_ref, f3b_ref, g1_ref, bb1_ref,
                       g2_ref, bb2_ref, o_ref, m_acc):
    i = pl.program_id(0)

    @pl.when(i == 0)
    def _():
        m_acc[...] = jnp.zeros_like(m_acc)

    acc = p0_ref[...].astype(jnp.float32) + p1_ref[...].astype(jnp.float32)
    deg = acc[:, dcol:dcol + 1]
    mean = acc / jnp.maximum(deg, 1.0)
    h2 = jnp.maximum(
        jnp.where(deg > 0, mean, y_ref[...].astype(jnp.float32)), 0.0)

    gids = gid_ref[0, 0, :]
    seg = lax.broadcasted_iota(jnp.int32, (16, h2.shape[0]), 0)
    oht = (gids[None, :] == seg).astype(jnp.float32)
    m_acc[...] += _dot_exact(oht, h2)

    @pl.when(i == nsteps - 1)
    def _():
        m = m_acc[...]
        cnt = m[:, dcol:dcol + 1]
        hg = m[:, :dcol] / jnp.maximum(cnt, 1.0)
        a = _dot(hg, wbil_ref[...]) * self_ref[...]
        z = _dot(a, f1w_ref[...]) + f1b_ref[...]
        mu = jnp.mean(z, axis=0, keepdims=True)
        var = jnp.mean((z - mu) ** 2, axis=0, keepdims=True)
        z = g1_ref[...] * (z - mu) / jnp.sqrt(var + 1e-5) + bb1_ref[...]
        z = jnp.maximum(z, 0.0)
        z = _dot(z, f2w_ref[...]) + f2b_ref[...]
        mu = jnp.mean(z, axis=0, keepdims=True)
        var = jnp.mean((z - mu) ** 2, axis=0, keepdims=True)
        z = g2_ref[...] * (z - mu) / jnp.sqrt(var + 1e-5) + bb2_ref[...]
        z = jnp.maximum(z, 0.0)
        o_ref[...] = _dot(z, f3w_ref[...]) + f3b_ref[...]


def _full(shape):
    return pl.BlockSpec(shape, lambda i: tuple(0 for _ in shape))


def kernel(x, edge_index, graph_ids, self_feat, W1, b1, W2, b2, Wbil,
           fc1_W, fc1_b, fc2_W, fc2_b, fc3_W, fc3_b,
           bn1_g, bn1_b, bn2_g, bn2_b):
    n, din = x.shape
    e = edge_index.shape[1]
    d1 = W1.shape[1]
    d1p = ((d1 + 1 + 31) // 32) * 32
    d2 = W2.shape[1]
    d2p = ((d2 + 1 + 31) // 32) * 32
    agg_dt = jnp.bfloat16
    nblk = n // _ROW_BLK

    e4 = edge_index.astype(jnp.int32).reshape(2, 32, e // 32 // _EB, _EB)
    gid3 = graph_ids.astype(jnp.int32).reshape(nblk, 1, _ROW_BLK)

    row = lambda shape: pl.BlockSpec(shape, lambda i: (i, 0))

    y1 = pl.pallas_call(
        _mm_body,
        grid=(nblk,),
        in_specs=[row((_ROW_BLK, din)), _full((din, d1)), _full((1, d1))],
        out_specs=row((_ROW_BLK, d1p)),
        out_shape=jax.ShapeDtypeStruct((n, d1p), agg_dt),
    )(x, W1, b1.reshape(1, -1))

    p0, p1 = _sc_aggregate(n, e, d1p, agg_dt)(y1, e4)

    y2 = pl.pallas_call(
        functools.partial(_combine_mm_body, d1),
        grid=(nblk,),
        in_specs=[row((_ROW_BLK, d1p)), row((_ROW_BLK, d1p)),
                  row((_ROW_BLK, d1p)), _full((d1, d2)), _full((1, d2))],
        out_specs=row((_ROW_BLK, d2p)),
        out_shape=jax.ShapeDtypeStruct((n, d2p), agg_dt),
    )(p0, p1, y1, W2, b2.reshape(1, -1))

    q0, q1 = _sc_aggregate(n, e, d2p, agg_dt)(y2, e4)

    out = pl.pallas_call(
        functools.partial(_readout_head_body, nblk, d2),
        grid=(nblk,),
        in_specs=[row((_ROW_BLK, d2p)), row((_ROW_BLK, d2p)),
                  row((_ROW_BLK, d2p)),
                  pl.BlockSpec((1, 1, _ROW_BLK), lambda i: (i, 0, 0)),
                  _full(self_feat.shape), _full(Wbil.shape),
                  _full(fc1_W.shape), _full((1, fc1_b.shape[0])),
                  _full(fc2_W.shape), _full((1, fc2_b.shape[0])),
                  _full(fc3_W.shape), _full((1, fc3_b.shape[0])),
                  _full((1, bn1_g.shape[0])), _full((1, bn1_b.shape[0])),
                  _full((1, bn2_g.shape[0])), _full((1, bn2_b.shape[0]))],
        out_specs=_full((16, fc3_W.shape[1])),
        out_shape=jax.ShapeDtypeStruct((16, fc3_W.shape[1]), jnp.float32),
        scratch_shapes=[pltpu.VMEM((16, d2p), jnp.float32)],
    )(q0, q1, y2, gid3, self_feat, Wbil,
      fc1_W, fc1_b.reshape(1, -1), fc2_W, fc2_b.reshape(1, -1),
      fc3_W, fc3_b.reshape(1, -1),
      bn1_g.reshape(1, -1), bn1_b.reshape(1, -1),
      bn2_g.reshape(1, -1), bn2_b.reshape(1, -1))
    return out


# speedup vs baseline: 22.2764x; 1.0577x over previous
import functools

import jax
import jax.numpy as jnp
from jax import lax
from jax.experimental import pallas as pl
from jax.experimental.pallas import tpu as pltpu
from jax.experimental.pallas import tpu_sc as plsc

_EB = 80
_ROW_BLK = 2000


@functools.cache
def _sc_aggregate(n, e, d, dt=jnp.float32):
    info = plsc.get_sparse_core_info()
    nc, ns = info.num_cores, info.num_subcores
    nw = nc * ns
    steps = e // nw // _EB
    assert steps % 4 == 1
    rpt = (n // ns) // 8 * 8
    tail = n - rpt * ns
    zrows = 48
    lanes = 32 if dt == jnp.bfloat16 else 16
    mesh = plsc.VectorSubcoreMesh(core_axis_name="c", subcore_axis_name="s")

    @functools.partial(
        pl.kernel,
        out_type=(jax.ShapeDtypeStruct((n, d), dt),
                  jax.ShapeDtypeStruct((n, d), dt)),
        mesh=mesh,
        scratch_types=[
            pltpu.VMEM((steps, _EB), jnp.int32),
            pltpu.VMEM((steps, _EB), jnp.int32),
            pltpu.VMEM((_EB, d), dt),
            pltpu.VMEM((_EB, d), dt),
            pltpu.VMEM((_EB, d), dt),
            pltpu.VMEM((_EB, d), dt),
            pltpu.VMEM((zrows, d), dt),
            pltpu.VMEM_SHARED((n, d), dt),
            pltpu.SemaphoreType.DMA,
            pltpu.SemaphoreType.DMA,
            pltpu.SemaphoreType.DMA,
            pltpu.SemaphoreType.DMA,
            pltpu.SemaphoreType.DMA,
            pltpu.SemaphoreType.DMA,
            pltpu.SemaphoreType.DMA,
            pltpu.SemaphoreType.DMA,
        ],
        compiler_params=pltpu.CompilerParams(use_tc_tiling_on_sc=False),
    )
    def agg(y_hbm, e4_hbm, out0, out1,
            src_v, dst_v, r0, r1, r2, r3, zbuf, acc_sh,
            g0, g1, g2, g3, s0, s1, s2, s3):
        rows = (r0, r1, r2, r3)
        gsem = (g0, g1, g2, g3)
        ssem = (s0, s1, s2, s3)
        c = lax.axis_index("c")
        s = lax.axis_index("s")
        wid = c * ns + s

        pltpu.async_copy(e4_hbm.at[0, wid], src_v, gsem[0])
        pltpu.async_copy(e4_hbm.at[1, wid], dst_v, gsem[1])

        def _zrow(r, carry):
            for k in range(d // lanes):
                zbuf[r, pl.ds(k * lanes, lanes)] = jnp.zeros((lanes,), dt)
            return carry
        lax.fori_loop(0, zrows, _zrow, 0)

        def _zcp(k, carry):
            pltpu.sync_copy(zbuf, acc_sh.at[pl.ds(s * rpt + k * zrows, zrows)])
            return carry
        lax.fori_loop(0, rpt // zrows, _zcp, 0)

        @pl.when(s == ns - 1)
        def _():
            pltpu.sync_copy(zbuf.at[pl.ds(0, tail)],
                            acc_sh.at[pl.ds(ns * rpt, tail)])
        pltpu.make_async_copy(e4_hbm.at[0, wid], src_v, gsem[0]).wait()
        pltpu.make_async_copy(e4_hbm.at[1, wid], dst_v, gsem[1]).wait()
        plsc.subcore_barrier()

        def g_start(step, b):
            pltpu.async_copy(y_hbm.at[src_v.at[step]], rows[b], gsem[b])

        def g_wait(step, b):
            pltpu.make_async_copy(
                y_hbm.at[src_v.at[step]], rows[b], gsem[b]).wait()

        def s_start(step, b):
            pltpu.async_copy(
                rows[b], acc_sh.at[dst_v.at[step]], ssem[b], add=True)

        def s_wait(step, b):
            pltpu.make_async_copy(
                rows[b], acc_sh.at[dst_v.at[step]], ssem[b]).wait()

        for k in range(3):
            g_start(k, k)
        g_wait(0, 0)
        s_start(0, 0)
        g_start(3, 3)

        def _grp(g, carry):
            base = 1 + 4 * g
            for k in range(4):
                st = base + k
                b = (1 + k) % 4
                bn = k % 4
                g_wait(st, b)
                s_start(st, b)
                s_wait(st - 1, bn)
                g_start(st + 3, bn)
            return carry
        lax.fori_loop(0, (steps - 5) // 4, _grp, 0)

        t = steps - 4
        g_wait(t, (t + 0) % 4); s_start(t, t % 4)
        s_wait(t - 1, (t + 3) % 4); g_start(t + 3, (t + 3) % 4)
        for k in range(1, 4):
            g_wait(t + k, (t + k) % 4)
            s_start(t + k, (t + k) % 4)
        for k in range(0, 4):
            s_wait(t + k, (t + k) % 4)

        plsc.subcore_barrier()
        sl = pl.ds(s * rpt, rpt)
        sl_t = pl.ds(ns * rpt, tail)

        @pl.when(c == 0)
        def _():
            pltpu.sync_copy(acc_sh.at[sl], out0.at[sl])

            @pl.when(s == ns - 1)
            def _():
                pltpu.sync_copy(acc_sh.at[sl_t], out0.at[sl_t])

        @pl.when(c == 1)
        def _():
            pltpu.sync_copy(acc_sh.at[sl], out1.at[sl])

            @pl.when(s == ns - 1)
            def _():
                pltpu.sync_copy(acc_sh.at[sl_t], out1.at[sl_t])

    return agg


def _dot(a, b):
    return jnp.dot(a, b, preferred_element_type=jnp.float32,
                   precision=lax.Precision.DEFAULT)


def _dot_exact(a, b):
    return jnp.dot(a, b, preferred_element_type=jnp.float32,
                   precision=lax.Precision.HIGHEST)


def _pad_ones(z, width):
    rows = z.shape[0]
    ones = jnp.ones((rows, 1), jnp.float32)
    zeros = jnp.zeros((rows, width - z.shape[1] - 1), jnp.float32)
    return jnp.concatenate([z, ones, zeros], axis=1)


def _mm_body(x_ref, w_ref, b_ref, o_ref):
    y = _dot(x_ref[...], w_ref[...]) + b_ref[...]
    o_ref[...] = _pad_ones(y, o_ref.shape[1]).astype(o_ref.dtype)


def _combine_mm_body(dcol, p0_ref, p1_ref, y_ref, w_ref, b_ref, o_ref):
    acc = p0_ref[...].astype(jnp.float32) + p1_ref[...].astype(jnp.float32)
    deg = acc[:, dcol:dcol + 1]
    mean = acc / jnp.maximum(deg, 1.0)
    h = jnp.maximum(
        jnp.where(deg > 0, mean, y_ref[...].astype(jnp.float32)), 0.0)
    z = _dot(h[:, :dcol], w_ref[...]) + b_ref[...]
    o_ref[...] = _pad_ones(z, o_ref.shape[1]).astype(o_ref.dtype)


def _readout_head_body(nsteps, dcol, p0_ref, p1_ref, y_ref, gid_ref,
                       self_ref, wbil_ref, f1w_ref, f1b_ref, f2w_ref,
                       f2b_ref, f3w_ref, f3b_ref, g1_ref, bb1_ref,
                       g2_ref, bb2_ref, o_ref, m_acc):
    i = pl.program_id(0)

    @pl.when(i == 0)
    def _():
        m_acc[...] = jnp.zeros_like(m_acc)

    acc = p0_ref[...].astype(jnp.float32) + p1_ref[...].astype(jnp.float32)
    deg = acc[:, dcol:dcol + 1]
    mean = acc / jnp.maximum(deg, 1.0)
    h2 = jnp.maximum(
        jnp.where(deg > 0, mean, y_ref[...].astype(jnp.float32)), 0.0)

    gids = gid_ref[0, 0, :]
    seg = lax.broadcasted_iota(jnp.int32, (16, h2.shape[0]), 0)
    oht = (gids[None, :] == seg).astype(jnp.float32)
    m_acc[...] += _dot_exact(oht, h2)

    @pl.when(i == nsteps - 1)
    def _():
        m = m_acc[...]
        cnt = m[:, dcol:dcol + 1]
        hg = m[:, :dcol] / jnp.maximum(cnt, 1.0)
        a = _dot(hg, wbil_ref[...]) * self_ref[...]
        z = _dot(a, f1w_ref[...]) + f1b_ref[...]
        mu = jnp.mean(z, axis=0, keepdims=True)
        var = jnp.mean((z - mu) ** 2, axis=0, keepdims=True)
        z = g1_ref[...] * (z - mu) / jnp.sqrt(var + 1e-5) + bb1_ref[...]
        z = jnp.maximum(z, 0.0)
        z = _dot(z, f2w_ref[...]) + f2b_ref[...]
        mu = jnp.mean(z, axis=0, keepdims=True)
        var = jnp.mean((z - mu) ** 2, axis=0, keepdims=True)
        z = g2_ref[...] * (z - mu) / jnp.sqrt(var + 1e-5) + bb2_ref[...]
        z = jnp.maximum(z, 0.0)
        o_ref[...] = _dot(z, f3w_ref[...]) + f3b_ref[...]


def _full(shape):
    return pl.BlockSpec(shape, lambda i: tuple(0 for _ in shape))


def kernel(x, edge_index, graph_ids, self_feat, W1, b1, W2, b2, Wbil,
           fc1_W, fc1_b, fc2_W, fc2_b, fc3_W, fc3_b,
           bn1_g, bn1_b, bn2_g, bn2_b):
    n, din = x.shape
    e = edge_index.shape[1]
    d1 = W1.shape[1]
    d1p = ((d1 + 1 + 31) // 32) * 32
    d2 = W2.shape[1]
    d2p = ((d2 + 1 + 31) // 32) * 32
    agg_dt = jnp.bfloat16
    nblk = n // _ROW_BLK

    e4 = edge_index.astype(jnp.int32).reshape(2, 32, e // 32 // _EB, _EB)
    gid3 = graph_ids.astype(jnp.int32).reshape(nblk, 1, _ROW_BLK)

    row = lambda shape: pl.BlockSpec(shape, lambda i: (i, 0))

    y1 = pl.pallas_call(
        _mm_body,
        grid=(nblk,),
        in_specs=[row((_ROW_BLK, din)), _full((din, d1)), _full((1, d1))],
        out_specs=row((_ROW_BLK, d1p)),
        out_shape=jax.ShapeDtypeStruct((n, d1p), agg_dt),
    )(x, W1, b1.reshape(1, -1))

    p0, p1 = _sc_aggregate(n, e, d1p, agg_dt)(y1, e4)

    y2 = pl.pallas_call(
        functools.partial(_combine_mm_body, d1),
        grid=(nblk,),
        in_specs=[row((_ROW_BLK, d1p)), row((_ROW_BLK, d1p)),
                  row((_ROW_BLK, d1p)), _full((d1, d2)), _full((1, d2))],
        out_specs=row((_ROW_BLK, d2p)),
        out_shape=jax.ShapeDtypeStruct((n, d2p), agg_dt),
    )(p0, p1, y1, W2, b2.reshape(1, -1))

    q0, q1 = _sc_aggregate(n, e, d2p, agg_dt)(y2, e4)

    out = pl.pallas_call(
        functools.partial(_readout_head_body, nblk, d2),
        grid=(nblk,),
        in_specs=[row((_ROW_BLK, d2p)), row((_ROW_BLK, d2p)),
                  row((_ROW_BLK, d2p)),
                  pl.BlockSpec((1, 1, _ROW_BLK), lambda i: (i, 0, 0)),
                  _full(self_feat.shape), _full(Wbil.shape),
                  _full(fc1_W.shape), _full((1, fc1_b.shape[0])),
                  _full(fc2_W.shape), _full((1, fc2_b.shape[0])),
                  _full(fc3_W.shape), _full((1, fc3_b.shape[0])),
                  _full((1, bn1_g.shape[0])), _full((1, bn1_b.shape[0])),
                  _full((1, bn2_g.shape[0])), _full((1, bn2_b.shape[0]))],
        out_specs=_full((16, fc3_W.shape[1])),
        out_shape=jax.ShapeDtypeStruct((16, fc3_W.shape[1]), jnp.float32),
        scratch_shapes=[pltpu.VMEM((16, d2p), jnp.float32)],
    )(q0, q1, y2, gid3, self_feat, Wbil,
      fc1_W, fc1_b.reshape(1, -1), fc2_W, fc2_b.reshape(1, -1),
      fc3_W, fc3_b.reshape(1, -1),
      bn1_g.reshape(1, -1), bn1_b.reshape(1, -1),
      bn2_g.reshape(1, -1), bn2_b.reshape(1, -1))
    return out
